# Initial kernel scaffold; baseline (speedup 1.0000x reference)
import jax
import jax.numpy as jnp
from jax.experimental import pallas as pl


def kernel(x, edge_index, edge_attr, batch, action_attr, edge_fc1_w, edge_fc1_b, edge_fc2_w, edge_fc2_b, gcn1_w, gcn1_b, gcn2_w, gcn2_b, state_fc_w, state_fc_b, action_fc_w, action_fc_b, q_fc1_w, q_fc1_b, q_out_w, q_out_b):
    raise NotImplementedError("write your pallas kernel here")


# speedup vs baseline: 1.4146x; 1.4146x over previous
import jax
import jax.numpy as jnp
from jax import lax
from jax.experimental import pallas as pl
from jax.experimental.pallas import tpu as pltpu
from jax.experimental.pallas import tpu_sc as plsc

N_NODES = 50000
N_EDGES = 800000
NPAD = 51200
EPAD = 819200
PADE = EPAD - N_EDGES
KROWS = 400
ROWS_PER_TILE = NPAD // 16
NB = 2048
TE = 3200
CH = 40


def _k1_body(ea_ref, w1_ref, b1_ref, w2_ref, b2_ref, ef_ref):
  ea = ea_ref[...]
  h = jnp.maximum(jnp.dot(ea, w1_ref[...],
                          preferred_element_type=jnp.float32) + b1_ref[...], 0.0)
  o = jnp.maximum(jnp.dot(h, w2_ref[...],
                          preferred_element_type=jnp.float32) + b2_ref[...], 0.0)
  ef_ref[0] = o[:, :16]
  ef_ref[1] = jnp.concatenate([o[:, 16:30], jnp.zeros((TE, 2), jnp.float32)],
                              axis=1)


def _edge_mlp(edge_attr, w1, b1, w2, b2):
  grid = N_EDGES // TE
  return pl.pallas_call(
      _k1_body,
      grid=(grid,),
      in_specs=[
          pl.BlockSpec((TE, 16), lambda i: (i, 0)),
          pl.BlockSpec((16, 128), lambda i: (0, 0)),
          pl.BlockSpec((1, 128), lambda i: (0, 0)),
          pl.BlockSpec((128, 30), lambda i: (0, 0)),
          pl.BlockSpec((1, 30), lambda i: (0, 0)),
      ],
      out_specs=pl.BlockSpec((2, TE, 16), lambda i: (0, i, 0)),
      out_shape=jax.ShapeDtypeStruct((2, EPAD, 16), jnp.float32),
      name="edge_mlp",
  )(edge_attr, w1, b1, w2, b2)


def _k2_body(ef_hbm, sidx_hbm, cidx_hbm, out_hbm, cnt_hbm,
             sidx_v, cidx_v, rows_v, zrows_v, ones1_v, z128_v,
             tab_sh, cnt_sh, sem):
  c = lax.axis_index("c")
  s = lax.axis_index("s")

  def fill(i, _):
    zrows_v[i, :] = jnp.zeros((16,), jnp.float32)
    return 0
  lax.fori_loop(0, 128, fill, 0)
  for k in range(8):
    ones1_v[pl.ds(k * 16, 16)] = jnp.ones((16,), jnp.float32)
    z128_v[pl.ds(k * 16, 16)] = jnp.zeros((16,), jnp.float32)

  def zdma(i, _):
    off = s * ROWS_PER_TILE + i * 128
    pltpu.sync_copy(zrows_v, tab_sh.at[pl.ds(off, 128)])
    return 0
  lax.fori_loop(0, ROWS_PER_TILE // 128, zdma, 0)

  def wout(i, _):
    off = s * ROWS_PER_TILE + i * 128
    pltpu.sync_copy(zrows_v, out_hbm.at[c].at[pl.ds(off, 128)])
    pltpu.sync_copy(z128_v, cnt_hbm.at[c].at[pl.ds(off, 128)])
    return 0
  lax.fori_loop(0, ROWS_PER_TILE // 128, wout, 0)


def _edge_scatter(ef, sidx_row, cidx):
  mesh = plsc.VectorSubcoreMesh(core_axis_name="c", subcore_axis_name="s")
  f = pl.kernel(
      _k2_body,
      out_type=[
          jax.ShapeDtypeStruct((2, NPAD, 16), jnp.float32),
          jax.ShapeDtypeStruct((2, NPAD), jnp.float32),
      ],
      mesh=mesh,
      scratch_types=[
          pltpu.VMEM((CH, 128), jnp.int32),
          pltpu.VMEM((CH, 128), jnp.int32),
          pltpu.VMEM((128, 16), jnp.float32),
          pltpu.VMEM((128, 16), jnp.float32),
          pltpu.VMEM((128,), jnp.float32),
          pltpu.VMEM((128,), jnp.float32),
          pltpu.VMEM_SHARED((NPAD, 16), jnp.float32),
          pltpu.VMEM_SHARED((NPAD,), jnp.float32),
          pltpu.SemaphoreType.DMA,
      ],
      name="edge_scatter",
  )
  return f(ef, sidx_row, cidx)


def _make_conv_scatter(nq, qoff, name):
  def body(u_hbm, gidx_hbm, sidx_hbm, out_hbm,
           gidx_v, sidx_v, rows_v, zrows_v, tab_sh, sem):
    c = lax.axis_index("c")
    s = lax.axis_index("s")

    def fill(i, _):
      zrows_v[i, :] = jnp.zeros((16,), jnp.float32)
      return 0
    lax.fori_loop(0, 128, fill, 0)

    def zdma(i, _):
      off = s * ROWS_PER_TILE + i * 128
      pltpu.sync_copy(zrows_v, tab_sh.at[pl.ds(off, 128)])
      return 0
    lax.fori_loop(0, ROWS_PER_TILE // 128, zdma, 0)
    plsc.subcore_barrier()

    uc = u_hbm.at[qoff + c]

    def outer(co, _):
      pltpu.sync_copy(gidx_hbm.at[s].at[pl.ds(co * CH, CH)], gidx_v)
      pltpu.sync_copy(sidx_hbm.at[s].at[pl.ds(co * CH, CH)], sidx_v)

      def loop(j, _):
        pltpu.async_copy(uc.at[gidx_v.at[j]], rows_v, sem).wait()
        pltpu.sync_copy(rows_v, tab_sh.at[sidx_v.at[j]], add=True)
        return 0
      lax.fori_loop(0, CH, loop, 0)
      return 0
    lax.fori_loop(0, KROWS // CH, outer, 0)
    plsc.subcore_barrier()

    def wout(i, _):
      off = s * ROWS_PER_TILE + i * 128
      pltpu.sync_copy(tab_sh.at[pl.ds(off, 128)], rows_v)
      pltpu.sync_copy(rows_v, out_hbm.at[c].at[pl.ds(off, 128)])
      return 0
    lax.fori_loop(0, ROWS_PER_TILE // 128, wout, 0)

  mesh = plsc.VectorSubcoreMesh(core_axis_name="c", subcore_axis_name="s")

  def run(u, gidx, sidx):
    f = pl.kernel(
        body,
        out_type=[jax.ShapeDtypeStruct((2, NPAD, 16), jnp.float32)],
        mesh=mesh,
        scratch_types=[
            pltpu.VMEM((CH, 128), jnp.int32),
            pltpu.VMEM((CH, 128), jnp.int32),
            pltpu.VMEM((128, 16), jnp.float32),
            pltpu.VMEM((128, 16), jnp.float32),
            pltpu.VMEM_SHARED((NPAD, 16), jnp.float32),
            pltpu.SemaphoreType.DMA,
        ],
        compiler_params=pltpu.CompilerParams(use_tc_tiling_on_sc=False),
        name=name,
    )
    return f(u, gidx, sidx)[0]

  return run


_conv1_scatter = _make_conv_scatter(2, 0, "conv1_scatter")
_conv2_scatter_a = _make_conv_scatter(4, 0, "conv2_scatter_a")
_conv2_scatter_b = _make_conv_scatter(4, 2, "conv2_scatter_b")


def _k3_body(x_ref, es_ref, cnt_ref, u0_ref, dinv_ref):
  xa = x_ref[...]
  cnt_row = cnt_ref[0]
  cnt_col = cnt_ref[1]
  inv_cnt = 1.0 / jnp.maximum(cnt_row, 1.0)
  h0a = xa[:, :16] + es_ref[0] * inv_cnt
  h0b = (jnp.concatenate([xa[:, 16:30], jnp.zeros((NB, 2), jnp.float32)], 1)
         + es_ref[1] * inv_cnt)
  dinv = lax.rsqrt(cnt_col + 1.0)
  u0_ref[0] = h0a * dinv
  u0_ref[1] = h0b * dinv
  dinv_ref[...] = jnp.broadcast_to(dinv, (NB, 16))


def _node_prep(x, esum, cnt1):
  grid = NPAD // NB
  return pl.pallas_call(
      _k3_body,
      grid=(grid,),
      in_specs=[
          pl.BlockSpec((NB, 30), lambda i: (i, 0)),
          pl.BlockSpec((2, NB, 16), lambda i: (0, i, 0)),
          pl.BlockSpec((2, NB, 1), lambda i: (0, i, 0)),
      ],
      out_specs=[
          pl.BlockSpec((2, NB, 16), lambda i: (0, i, 0)),
          pl.BlockSpec((NB, 16), lambda i: (i, 0)),
      ],
      out_shape=[
          jax.ShapeDtypeStruct((2, NPAD, 16), jnp.float32),
          jax.ShapeDtypeStruct((NPAD, 16), jnp.float32),
      ],
      name="node_prep",
  )(x, esum, cnt1)


def _k5_body(scat_ref, u0_ref, dinv_ref, w1_ref, b1_ref, w2_ref, u1_ref):
  dinv = dinv_ref[...][:, 0:1]
  agg_a = dinv * (scat_ref[0] + u0_ref[0])
  agg_b = dinv * (scat_ref[1] + u0_ref[1])
  agg = jnp.concatenate([agg_a, agg_b[:, :14]], axis=1)
  h1 = jnp.maximum(jnp.dot(agg, w1_ref[...],
                           preferred_element_type=jnp.float32) + b1_ref[...],
                   0.0)
  u1 = jnp.dot(h1, w2_ref[...], preferred_element_type=jnp.float32) * dinv
  for q in range(4):
    u1_ref[q] = u1[:, q * 16:(q + 1) * 16]


def _conv1_dense(scat1, u0, dinvw, w1, b1, w2):
  grid = NPAD // NB
  return pl.pallas_call(
      _k5_body,
      grid=(grid,),
      in_specs=[
          pl.BlockSpec((2, NB, 16), lambda i: (0, i, 0)),
          pl.BlockSpec((2, NB, 16), lambda i: (0, i, 0)),
          pl.BlockSpec((NB, 16), lambda i: (i, 0)),
          pl.BlockSpec((30, 128), lambda i: (0, 0)),
          pl.BlockSpec((1, 128), lambda i: (0, 0)),
          pl.BlockSpec((128, 64), lambda i: (0, 0)),
      ],
      out_specs=pl.BlockSpec((4, NB, 16), lambda i: (0, i, 0)),
      out_shape=jax.ShapeDtypeStruct((4, NPAD, 16), jnp.float32),
      name="conv1_dense",
  )(scat1, u0, dinvw, w1, b1, w2)


def _k7_body(sa_ref, sb_ref, u1_ref, dinv_ref, b2_ref, aa_ref, sw_ref,
             sb2_ref, aw_ref, ab_ref, qw_ref, qb_ref, ow_ref, ob_ref,
             out_ref, acc_ref):
  i = pl.program_id(0)

  @pl.when(i == 0)
  def _():
    acc_ref[...] = jnp.zeros((1, 64), jnp.float32)

  dinv = dinv_ref[...][:, 0:1]
  h2 = jnp.concatenate(
      [dinv * (sa_ref[0] + u1_ref[0]), dinv * (sa_ref[1] + u1_ref[1]),
       dinv * (sb_ref[0] + u1_ref[2]), dinv * (sb_ref[1] + u1_ref[3])],
      axis=1)
  h2 = jnp.maximum(h2 + b2_ref[...], 0.0)
  gid = i * NB + lax.broadcasted_iota(jnp.int32, (NB, 1), 0)
  h2 = jnp.where(gid < N_NODES, h2, 0.0)
  acc_ref[...] += jnp.sum(h2, axis=0, keepdims=True)

  @pl.when(i == (NPAD // NB) - 1)
  def _():
    g = acc_ref[...] * (1.0 / N_NODES)
    sf = jnp.maximum(jnp.dot(g, sw_ref[...],
                             preferred_element_type=jnp.float32) + sb2_ref[...],
                     0.0)
    af = jnp.maximum(jnp.dot(aa_ref[...], aw_ref[...],
                             preferred_element_type=jnp.float32) + ab_ref[...],
                     0.0)
    comb = jnp.concatenate([sf, af], axis=1)
    q = jnp.maximum(jnp.dot(comb, qw_ref[...],
                            preferred_element_type=jnp.float32) + qb_ref[...],
                    0.0)
    out_ref[...] = jnp.dot(q, ow_ref[...],
                           preferred_element_type=jnp.float32) + ob_ref[...]


def _pool_head(s2a, s2b, u1, dinvw, b2, aa, sw, sb, aw, ab, qw, qb, ow, ob):
  grid = NPAD // NB
  full = lambda a: pl.BlockSpec(a.shape, lambda i: (0,) * a.ndim)
  return pl.pallas_call(
      _k7_body,
      grid=(grid,),
      in_specs=[
          pl.BlockSpec((2, NB, 16), lambda i: (0, i, 0)),
          pl.BlockSpec((2, NB, 16), lambda i: (0, i, 0)),
          pl.BlockSpec((4, NB, 16), lambda i: (0, i, 0)),
          pl.BlockSpec((NB, 16), lambda i: (i, 0)),
          full(b2), full(aa), full(sw), full(sb), full(aw), full(ab),
          full(qw), full(qb), full(ow), full(ob),
      ],
      out_specs=pl.BlockSpec((1, 1), lambda i: (0, 0)),
      out_shape=jax.ShapeDtypeStruct((1, 1), jnp.float32),
      scratch_shapes=[pltpu.VMEM((1, 64), jnp.float32)],
      name="pool_head",
  )(s2a, s2b, u1, dinvw, b2, aa, sw, sb, aw, ab, qw, qb, ow, ob)


def kernel(x, edge_index, edge_attr, batch, action_attr,
           edge_fc1_w, edge_fc1_b, edge_fc2_w, edge_fc2_b,
           gcn1_w, gcn1_b, gcn2_w, gcn2_b,
           state_fc_w, state_fc_b, action_fc_w, action_fc_b,
           q_fc1_w, q_fc1_b, q_out_w, q_out_b):
  row = edge_index[0].astype(jnp.int32)
  col = edge_index[1].astype(jnp.int32)

  pad_g = jnp.zeros((PADE,), jnp.int32)
  pad_s = N_NODES + (lax.iota(jnp.int32, PADE) % 1024)
  shape3 = (16, KROWS, 128)
  g_row = jnp.concatenate([row, pad_g]).reshape(shape3)
  s_row = jnp.concatenate([row, pad_s]).reshape(shape3)
  s_col = jnp.concatenate([col, pad_s]).reshape(shape3)
  cidx = jnp.concatenate([s_row, s_col]).reshape(32, KROWS, 128)

  ef = _edge_mlp(edge_attr, edge_fc1_w, edge_fc1_b.reshape(1, 128),
                 edge_fc2_w, edge_fc2_b.reshape(1, 30))
  esum = jnp.stack([
      jax.ops.segment_sum(ef[0, :N_EDGES], row, num_segments=NPAD),
      jax.ops.segment_sum(ef[1, :N_EDGES], row, num_segments=NPAD)])
  ones_e = jnp.ones((N_EDGES,), jnp.float32)
  cnt = jnp.stack([jax.ops.segment_sum(ones_e, row, num_segments=NPAD),
                   jax.ops.segment_sum(ones_e, col, num_segments=NPAD)])
  u0, dinvw = _node_prep(x, esum, cnt.reshape(2, NPAD, 1))

  def _jscat(u, q):
    return jax.ops.segment_sum(u[q][row], col, num_segments=NPAD)
  scat1 = jnp.stack([_jscat(u0, 0), _jscat(u0, 1)])
  u1 = _conv1_dense(scat1, u0, dinvw, gcn1_w, gcn1_b.reshape(1, 128), gcn2_w)
  s2a = jnp.stack([_jscat(u1, 0), _jscat(u1, 1)])
  s2b = jnp.stack([_jscat(u1, 2), _jscat(u1, 3)])
  return _pool_head(s2a, s2b, u1, dinvw, gcn2_b.reshape(1, 64), action_attr,
                    state_fc_w, state_fc_b.reshape(1, 64),
                    action_fc_w, action_fc_b.reshape(1, 64),
                    q_fc1_w, q_fc1_b.reshape(1, 64),
                    q_out_w, q_out_b.reshape(1, 1))
